# baseline (speedup 1.0000x reference)
import functools

import jax
import jax.numpy as jnp
from jax import lax
from jax.experimental import pallas as pl
from jax.experimental.pallas import tpu as pltpu
from jax.experimental.pallas import tpu_sc as plsc

VOCAB = 100000
HIDDEN = 1024
B = 4
S = 2048
EPS = 1e-5

NC = 2
NS = 16
NW = NC * NS
SEQ_PER_W = S // NW
SUB = 16
NSUB = SEQ_PER_W // SUB
NTOK = B * SUB
NVREG = HIDDEN // 16


def _lane_allreduce_sum(a):
    dnums = lax.GatherDimensionNumbers(
        offset_dims=(), collapsed_slice_dims=(0,), start_index_map=(0,))
    for k in (8, 4, 2, 1):
        idx = lax.iota(jnp.int32, 16) ^ k
        a = a + lax.gather(a, idx[:, None], dnums, slice_sizes=(1,),
                           mode=lax.GatherScatterMode.PROMISE_IN_BOUNDS)
    return a


def _sc_body(ids_hbm, word_hbm, pos_hbm, gam_hbm, bet_hbm, out_hbm,
             ids_v, pidx_v, pos_v, rows_v, gam_v, bet_v, sem):
    wid = lax.axis_index("s") * NC + lax.axis_index("c")
    seq_base = wid * SEQ_PER_W
    pltpu.sync_copy(gam_hbm, gam_v)
    pltpu.sync_copy(bet_hbm, bet_v)

    for sub in range(NSUB):
        g0 = seq_base + sub * SUB
        for b in range(B):
            pltpu.sync_copy(ids_hbm.at[pl.ds(b * S + g0, SUB)],
                            ids_v.at[pl.ds(b * SUB, SUB)])
        pidx_v[...] = lax.iota(jnp.int32, 16) + (g0 + 2)
        pltpu.async_copy(pos_hbm.at[pidx_v], pos_v, sem).wait()
        pltpu.async_copy(word_hbm.at[ids_v], rows_v, sem).wait()

        def token_body(t, carry):
            srow = lax.rem(t, SUB)
            zeros = jnp.zeros((16,), jnp.float32)

            def p1(j, acc):
                a1, a2 = acc
                w = rows_v[t, pl.ds(j * 16, 16)]
                p = pos_v[srow, pl.ds(j * 16, 16)]
                e = w + p
                rows_v[t, pl.ds(j * 16, 16)] = e
                return (a1 + e, a2 + e * e)

            a1, a2 = lax.fori_loop(0, NVREG, p1, (zeros, zeros))
            s1 = _lane_allreduce_sum(a1)
            s2 = _lane_allreduce_sum(a2)
            mean = s1 * (1.0 / HIDDEN)
            var = s2 * (1.0 / HIDDEN) - mean * mean
            x = var + EPS
            iu = lax.bitcast_convert_type(x, jnp.uint32)
            iu = jnp.full((16,), 0x5F3759DF, jnp.uint32) - (
                lax.shift_right_logical(iu, jnp.full((16,), 1, jnp.uint32)))
            y = lax.bitcast_convert_type(iu, jnp.float32)
            y = y * (1.5 - 0.5 * x * y * y)
            y = y * (1.5 - 0.5 * x * y * y)
            y = y * (1.5 - 0.5 * x * y * y)

            def p2(j, c):
                e = rows_v[t, pl.ds(j * 16, 16)]
                g = gam_v[pl.ds(j * 16, 16)]
                bb = bet_v[pl.ds(j * 16, 16)]
                rows_v[t, pl.ds(j * 16, 16)] = (e - mean) * y * g + bb
                return c

            lax.fori_loop(0, NVREG, p2, 0)
            return carry

        lax.fori_loop(0, NTOK, token_body, 0)

        for b in range(B):
            pltpu.sync_copy(rows_v.at[pl.ds(b * SUB, SUB)],
                            out_hbm.at[pl.ds(b * S + g0, SUB)])


@jax.jit
def _sc_call(ids_flat, word_table, pos_table, ln_gamma, ln_beta):
    mesh = plsc.VectorSubcoreMesh(core_axis_name="c", subcore_axis_name="s")
    f = functools.partial(
        pl.kernel,
        mesh=mesh,
        out_type=jax.ShapeDtypeStruct((B * S, HIDDEN), jnp.float32),
        scratch_types=[
            pltpu.VMEM((NTOK,), jnp.int32),
            pltpu.VMEM((SUB,), jnp.int32),
            pltpu.VMEM((SUB, HIDDEN), jnp.float32),
            pltpu.VMEM((NTOK, HIDDEN), jnp.float32),
            pltpu.VMEM((HIDDEN,), jnp.float32),
            pltpu.VMEM((HIDDEN,), jnp.float32),
            pltpu.SemaphoreType.DMA,
        ],
    )(_sc_body)
    return f(ids_flat, word_table, pos_table, ln_gamma, ln_beta)


def kernel(input_ids, word_table, pos_table, ln_gamma, ln_beta):
    ids_flat = input_ids.reshape(-1)
    out = _sc_call(ids_flat, word_table, pos_table, ln_gamma, ln_beta)
    return out.reshape(B, S, HIDDEN)


# speedup vs baseline: 1.1868x; 1.1868x over previous
import functools

import jax
import jax.numpy as jnp
from jax import lax
from jax.experimental import pallas as pl
from jax.experimental.pallas import tpu as pltpu
from jax.experimental.pallas import tpu_sc as plsc

VOCAB = 100000
HIDDEN = 1024
B = 4
S = 2048
EPS = 1e-5

NC = 2
NS = 16
NW = NC * NS
SEQ_PER_W = S // NW
SUB = 16
NSUB = SEQ_PER_W // SUB
NTOK = B * SUB
NVREG = HIDDEN // 16


def _lane_allreduce_sum(a):
    dnums = lax.GatherDimensionNumbers(
        offset_dims=(), collapsed_slice_dims=(0,), start_index_map=(0,))
    for k in (8, 4, 2, 1):
        idx = lax.iota(jnp.int32, 16) ^ k
        a = a + lax.gather(a, idx[:, None], dnums, slice_sizes=(1,),
                           mode=lax.GatherScatterMode.PROMISE_IN_BOUNDS)
    return a


def _sc_body(ids_hbm, word_hbm, pos_hbm, gam_hbm, bet_hbm, out_hbm,
             ids_v, pidx_v, pos_v, rows_v, gam_v, bet_v, sem):
    wid = lax.axis_index("s") * NC + lax.axis_index("c")
    seq_base = wid * SEQ_PER_W
    pltpu.sync_copy(gam_hbm, gam_v)
    pltpu.sync_copy(bet_hbm, bet_v)

    for sub in range(NSUB):
        g0 = seq_base + sub * SUB
        for b in range(B):
            pltpu.sync_copy(ids_hbm.at[pl.ds(b * S + g0, SUB)],
                            ids_v.at[pl.ds(b * SUB, SUB)])
        pidx_v[...] = lax.iota(jnp.int32, 16) + (g0 + 2)
        pltpu.async_copy(pos_hbm.at[pidx_v], pos_v, sem).wait()
        pltpu.async_copy(word_hbm.at[ids_v], rows_v, sem).wait()

        def token_body(t, carry):
            srow = lax.rem(t, SUB)
            zeros = jnp.zeros((16,), jnp.float32)
            U = 8

            def p1(jj, acc):
                a1, a2 = acc
                base = jj * (16 * U)
                for u in range(U):
                    off = base + u * 16
                    w = rows_v[t, pl.ds(off, 16)]
                    p = pos_v[srow, pl.ds(off, 16)]
                    e = w + p
                    rows_v[t, pl.ds(off, 16)] = e
                    a1 = a1 + e
                    a2 = a2 + e * e
                return (a1, a2)

            a1, a2 = lax.fori_loop(0, NVREG // U, p1, (zeros, zeros))
            s1 = _lane_allreduce_sum(a1)
            s2 = _lane_allreduce_sum(a2)
            mean = s1 * (1.0 / HIDDEN)
            var = s2 * (1.0 / HIDDEN) - mean * mean
            x = var + EPS
            iu = lax.bitcast_convert_type(x, jnp.uint32)
            iu = jnp.full((16,), 0x5F3759DF, jnp.uint32) - (
                lax.shift_right_logical(iu, jnp.full((16,), 1, jnp.uint32)))
            y = lax.bitcast_convert_type(iu, jnp.float32)
            y = y * (1.5 - 0.5 * x * y * y)
            y = y * (1.5 - 0.5 * x * y * y)

            def p2(jj, c):
                base = jj * (16 * U)
                for u in range(U):
                    off = base + u * 16
                    e = rows_v[t, pl.ds(off, 16)]
                    g = gam_v[pl.ds(off, 16)]
                    bb = bet_v[pl.ds(off, 16)]
                    rows_v[t, pl.ds(off, 16)] = (e - mean) * y * g + bb
                return c

            lax.fori_loop(0, NVREG // U, p2, 0)
            return carry

        lax.fori_loop(0, NTOK, token_body, 0)

        for b in range(B):
            pltpu.sync_copy(rows_v.at[pl.ds(b * SUB, SUB)],
                            out_hbm.at[pl.ds(b * S + g0, SUB)])


@jax.jit
def _sc_call(ids_flat, word_table, pos_table, ln_gamma, ln_beta):
    mesh = plsc.VectorSubcoreMesh(core_axis_name="c", subcore_axis_name="s")
    f = functools.partial(
        pl.kernel,
        mesh=mesh,
        out_type=jax.ShapeDtypeStruct((B * S, HIDDEN), jnp.float32),
        scratch_types=[
            pltpu.VMEM((NTOK,), jnp.int32),
            pltpu.VMEM((SUB,), jnp.int32),
            pltpu.VMEM((SUB, HIDDEN), jnp.float32),
            pltpu.VMEM((NTOK, HIDDEN), jnp.float32),
            pltpu.VMEM((HIDDEN,), jnp.float32),
            pltpu.VMEM((HIDDEN,), jnp.float32),
            pltpu.SemaphoreType.DMA,
        ],
    )(_sc_body)
    return f(ids_flat, word_table, pos_table, ln_gamma, ln_beta)


def kernel(input_ids, word_table, pos_table, ln_gamma, ln_beta):
    ids_flat = input_ids.reshape(-1)
    out = _sc_call(ids_flat, word_table, pos_table, ln_gamma, ln_beta)
    return out.reshape(B, S, HIDDEN)
